# baseline (speedup 1.0000x reference)
import jax
import jax.numpy as jnp
from jax import lax
from jax.experimental import pallas as pl
from jax.experimental.pallas import tpu as pltpu

_VS = (384, 96, 24, 6)
_CONVS = (
    ("enc0", 1, 2, 0),
    ("enc1", 2, 4, 1),
    ("enc2", 4, 8, 2),
    ("bottom", 8, 16, 3),
    ("dec2", 24, 8, 2),
    ("dec1", 12, 4, 1),
    ("dec0", 6, 2, 0),
    ("final", 2, 1, 0),
)

_BB = 1024


def _dot(a, b):
    return lax.dot(a, b, preferred_element_type=jnp.float32,
                   precision=lax.Precision.HIGHEST)


def _build_lap(nn_ref, nnt_ref, dcol_ref, drow_ref, v, k):
    lane = lax.broadcasted_iota(jnp.int32, (v, v), 1)
    sub = lax.broadcasted_iota(jnp.int32, (v, v), 0)
    b = (lane == nn_ref[:, 0:1]).astype(jnp.float32)
    bt = (sub == nnt_ref[0:1, :]).astype(jnp.float32)
    for j in range(1, k):
        b = b + (lane == nn_ref[:, j:j + 1]).astype(jnp.float32)
        bt = bt + (sub == nnt_ref[j:j + 1, :]).astype(jnp.float32)
    return (-dcol_ref[...]) * (b + bt) * drow_ref[...]


def _pool_mat(v):
    lane = lax.broadcasted_iota(jnp.int32, (v // 4, v), 1)
    sub = lax.broadcasted_iota(jnp.int32, (v // 4, v), 0)
    return jnp.where(lane // 4 == sub, 0.25, 0.0).astype(jnp.float32)


def _unpool_mat(v):
    lane = lax.broadcasted_iota(jnp.int32, (v, v // 4), 1)
    sub = lax.broadcasted_iota(jnp.int32, (v, v // 4), 0)
    return jnp.where(sub // 4 == lane, 1.0, 0.0).astype(jnp.float32)


def _mix(hs, th_ref, kk, f):
    acc = th_ref[kk, 0, f] * hs[0]
    for c in range(1, len(hs)):
        acc = acc + th_ref[kk, c, f] * hs[c]
    return acc


def _conv_lfirst(lap, hs, th_ref, b_ref, cout):
    t1 = [_dot(lap, h) for h in hs]
    t2 = [2.0 * _dot(lap, a) - h for a, h in zip(t1, hs)]
    return [
        _mix(hs, th_ref, 0, f) + _mix(t1, th_ref, 1, f)
        + _mix(t2, th_ref, 2, f) + b_ref[f]
        for f in range(cout)
    ]


def _conv_mixfirst(lap, hs, th_ref, b_ref, cout):
    outs = []
    for f in range(cout):
        m0 = _mix(hs, th_ref, 0, f)
        m1 = _mix(hs, th_ref, 1, f)
        m2 = _mix(hs, th_ref, 2, f)
        outs.append(m0 - m2 + _dot(lap, m1 + 2.0 * _dot(lap, m2)) + b_ref[f])
    return outs


def _relu(hs):
    return [jnp.maximum(h, 0.0) for h in hs]


def _body(*refs):
    x_ref = refs[0]
    g = {}
    p = 1
    ks = []
    for l, v in enumerate(_VS):
        nn_ref, nnt_ref, dcol_ref, drow_ref = refs[p:p + 4]
        p += 4
        k = nn_ref.shape[1]
        ks.append(k)
        g[l] = (nn_ref, nnt_ref, dcol_ref, drow_ref)
    th = {}
    for name, cin, cout, lvl in _CONVS:
        th[name] = (refs[p], refs[p + 1])
        p += 2
    out_ref = refs[p]

    laps = [_build_lap(*g[l], _VS[l], ks[l]) for l in range(4)]
    pools = [_pool_mat(_VS[l]) for l in range(3)]
    unpools = [_unpool_mat(_VS[l]) for l in range(3)]

    h0 = x_ref[...].T

    e0 = _relu(_conv_lfirst(laps[0], [h0], *th["enc0"], 2))
    e1 = _relu(_conv_lfirst(laps[1], [_dot(pools[0], h) for h in e0],
                            *th["enc1"], 4))
    e2 = _relu(_conv_lfirst(laps[2], [_dot(pools[1], h) for h in e1],
                            *th["enc2"], 8))
    bb = _relu(_conv_lfirst(laps[3], [_dot(pools[2], h) for h in e2],
                            *th["bottom"], 16))
    d2 = _relu(_conv_mixfirst(laps[2], [_dot(unpools[2], h) for h in bb] + e2,
                              *th["dec2"], 8))
    d1 = _relu(_conv_mixfirst(laps[1], [_dot(unpools[1], h) for h in d2] + e1,
                              *th["dec1"], 4))
    d0 = _relu(_conv_mixfirst(laps[0], [_dot(unpools[0], h) for h in d1] + e0,
                              *th["dec0"], 2))
    fin = _conv_mixfirst(laps[0], d0, *th["final"], 1)

    out_ref[...] = fin[0].T


def kernel(x, params, graphs):
    batch, v0 = x.shape
    nb = batch // _BB

    args = [x]
    in_specs = [pl.BlockSpec((_BB, v0), lambda i: (i, 0))]

    def _full(a):
        args.append(a)
        in_specs.append(pl.BlockSpec(a.shape, lambda i: (0,) * a.ndim))

    for l, v in enumerate(_VS):
        dst = graphs["dst%d" % l]
        dinv = graphs["dinv%d" % l]
        e = dst.shape[0]
        k = e // (2 * v)
        nn = dst[: e // 2].reshape(v, k)
        _full(nn)
        _full(nn.T)
        _full(dinv.reshape(v, 1))
        _full(dinv.reshape(1, v))

    for name, cin, cout, lvl in _CONVS:
        for a in (params[name + "_theta"], params[name + "_bias"]):
            args.append(a)
            in_specs.append(pl.BlockSpec(memory_space=pltpu.SMEM))

    out = pl.pallas_call(
        _body,
        grid=(nb,),
        in_specs=in_specs,
        out_specs=pl.BlockSpec((_BB, v0), lambda i: (i, 0)),
        out_shape=jax.ShapeDtypeStruct((batch, v0), jnp.float32),
        compiler_params=pltpu.CompilerParams(
            dimension_semantics=("arbitrary",),
        ),
    )(*args)
    return out


# speedup vs baseline: 1.7480x; 1.7480x over previous
import jax
import jax.numpy as jnp
from jax import lax
from jax.experimental import pallas as pl
from jax.experimental.pallas import tpu as pltpu

_VS = (384, 96, 24, 6)
_CONVS = (
    ("enc0", 1, 2, 0),
    ("enc1", 2, 4, 1),
    ("enc2", 4, 8, 2),
    ("bottom", 8, 16, 3),
    ("dec2", 24, 8, 2),
    ("dec1", 12, 4, 1),
    ("dec0", 6, 2, 0),
    ("final", 2, 1, 0),
)

_BB = 512


def _split(a):
    hi = a.astype(jnp.bfloat16)
    lo = (a - hi.astype(jnp.float32)).astype(jnp.bfloat16)
    return hi, lo


def _dot(m, h):
    mhi, mlo = m
    hh, hl = _split(h)
    acc = lax.dot(mhi, hh, preferred_element_type=jnp.float32)
    acc = acc + lax.dot(mhi, hl, preferred_element_type=jnp.float32)
    if mlo is not None:
        acc = acc + lax.dot(mlo, hh, preferred_element_type=jnp.float32)
    return acc


def _build_lap(nn_ref, nnt_ref, dcol_ref, drow_ref, v, k):
    lane = lax.broadcasted_iota(jnp.int32, (v, v), 1)
    sub = lax.broadcasted_iota(jnp.int32, (v, v), 0)
    b = (lane == nn_ref[:, 0:1]).astype(jnp.float32)
    bt = (sub == nnt_ref[0:1, :]).astype(jnp.float32)
    for j in range(1, k):
        b = b + (lane == nn_ref[:, j:j + 1]).astype(jnp.float32)
        bt = bt + (sub == nnt_ref[j:j + 1, :]).astype(jnp.float32)
    return (-dcol_ref[...]) * (b + bt) * drow_ref[...]


def _pool_mat(v):
    lane = lax.broadcasted_iota(jnp.int32, (v // 4, v), 1)
    sub = lax.broadcasted_iota(jnp.int32, (v // 4, v), 0)
    return jnp.where(lane // 4 == sub, 0.25, 0.0).astype(jnp.float32)


def _unpool_mat(v):
    lane = lax.broadcasted_iota(jnp.int32, (v, v // 4), 1)
    sub = lax.broadcasted_iota(jnp.int32, (v, v // 4), 0)
    return jnp.where(sub // 4 == lane, 1.0, 0.0).astype(jnp.float32)


def _mix(hs, th_ref, kk, f):
    acc = th_ref[kk, 0, f] * hs[0]
    for c in range(1, len(hs)):
        acc = acc + th_ref[kk, c, f] * hs[c]
    return acc


def _conv_lfirst(lap, hs, th_ref, b_ref, cout):
    t1 = [_dot(lap, h) for h in hs]
    t2 = [2.0 * _dot(lap, a) - h for a, h in zip(t1, hs)]
    return [
        _mix(hs, th_ref, 0, f) + _mix(t1, th_ref, 1, f)
        + _mix(t2, th_ref, 2, f) + b_ref[f]
        for f in range(cout)
    ]


def _conv_mixfirst(lap, hs, th_ref, b_ref, cout):
    outs = []
    for f in range(cout):
        m0 = _mix(hs, th_ref, 0, f)
        m1 = _mix(hs, th_ref, 1, f)
        m2 = _mix(hs, th_ref, 2, f)
        outs.append(m0 - m2 + _dot(lap, m1 + 2.0 * _dot(lap, m2)) + b_ref[f])
    return outs


def _relu(hs):
    return [jnp.maximum(h, 0.0) for h in hs]


def _body(*refs):
    x_ref = refs[0]
    g = {}
    p = 1
    ks = []
    for l, v in enumerate(_VS):
        nn_ref, nnt_ref, dcol_ref, drow_ref = refs[p:p + 4]
        p += 4
        k = nn_ref.shape[1]
        ks.append(k)
        g[l] = (nn_ref, nnt_ref, dcol_ref, drow_ref)
    th = {}
    for name, cin, cout, lvl in _CONVS:
        th[name] = (refs[p], refs[p + 1])
        p += 2
    out_ref = refs[p]

    laps = [_split(_build_lap(*g[l], _VS[l], ks[l])) for l in range(4)]
    pools = [(_pool_mat(_VS[l]).astype(jnp.bfloat16), None) for l in range(3)]
    unpools = [(_unpool_mat(_VS[l]).astype(jnp.bfloat16), None)
               for l in range(3)]

    h0 = x_ref[...].T

    e0 = _relu(_conv_lfirst(laps[0], [h0], *th["enc0"], 2))
    e1 = _relu(_conv_lfirst(laps[1], [_dot(pools[0], h) for h in e0],
                            *th["enc1"], 4))
    e2 = _relu(_conv_lfirst(laps[2], [_dot(pools[1], h) for h in e1],
                            *th["enc2"], 8))
    bb = _relu(_conv_lfirst(laps[3], [_dot(pools[2], h) for h in e2],
                            *th["bottom"], 16))
    d2 = _relu(_conv_mixfirst(laps[2], [_dot(unpools[2], h) for h in bb] + e2,
                              *th["dec2"], 8))
    d1 = _relu(_conv_mixfirst(laps[1], [_dot(unpools[1], h) for h in d2] + e1,
                              *th["dec1"], 4))
    d0 = _relu(_conv_mixfirst(laps[0], [_dot(unpools[0], h) for h in d1] + e0,
                              *th["dec0"], 2))
    fin = _conv_mixfirst(laps[0], d0, *th["final"], 1)

    out_ref[...] = fin[0].T


def kernel(x, params, graphs):
    batch, v0 = x.shape
    nb = batch // _BB

    args = [x]
    in_specs = [pl.BlockSpec((_BB, v0), lambda i: (i, 0))]

    def _full(a):
        args.append(a)
        in_specs.append(pl.BlockSpec(a.shape, lambda i: (0,) * a.ndim))

    for l, v in enumerate(_VS):
        dst = graphs["dst%d" % l]
        dinv = graphs["dinv%d" % l]
        e = dst.shape[0]
        k = e // (2 * v)
        nn = dst[: e // 2].reshape(v, k)
        _full(nn)
        _full(nn.T)
        _full(dinv.reshape(v, 1))
        _full(dinv.reshape(1, v))

    for name, cin, cout, lvl in _CONVS:
        for a in (params[name + "_theta"], params[name + "_bias"]):
            args.append(a)
            in_specs.append(pl.BlockSpec(memory_space=pltpu.SMEM))

    out = pl.pallas_call(
        _body,
        grid=(nb,),
        in_specs=in_specs,
        out_specs=pl.BlockSpec((_BB, v0), lambda i: (i, 0)),
        out_shape=jax.ShapeDtypeStruct((batch, v0), jnp.float32),
        compiler_params=pltpu.CompilerParams(
            dimension_semantics=("arbitrary",),
        ),
    )(*args)
    return out
